# Initial kernel scaffold; baseline (speedup 1.0000x reference)
import jax
import jax.numpy as jnp
from jax.experimental import pallas as pl


def kernel(localizations, classifications, localizations_default):
    raise NotImplementedError("write your pallas kernel here")


# speedup vs baseline: 242.3714x; 242.3714x over previous
import functools

import jax
import jax.numpy as jnp
from jax import lax
from jax.experimental import pallas as pl
from jax.experimental.pallas import tpu as pltpu, tpu_sc as plsc

_N = 20000
_P = 20480
_NCH = _P // 16
_B = 16
_K = 200
_KPAD = 208
_THR = 0.7
_OVR = 0.5


def _dense_body(cls_ref, loc_ref, dft_ref, sc_ref, st_ref, en_ref):
    x0 = cls_ref[0, 0]
    x1 = cls_ref[0, 1]
    x2 = cls_ref[0, 2]
    m = jnp.maximum(x0, jnp.maximum(x1, x2))
    e0 = jnp.exp(x0 - m)
    e1 = jnp.exp(x1 - m)
    e2 = jnp.exp(x2 - m)
    s = e0 + e1 + e2
    sc_ref[0, 0] = e1 / s
    sc_ref[0, 1] = e2 / s
    l0 = loc_ref[0, 0]
    l1 = loc_ref[0, 1]
    d0 = dft_ref[0]
    d1 = dft_ref[1]
    centers = d0 + 0.1 * l0 * d1
    widths = d1 * jnp.exp(0.2 * l1)
    st = centers - widths / 2.0
    st_ref[0] = st
    en_ref[0] = st + widths


_dense = pl.pallas_call(
    _dense_body,
    grid=(_B,),
    in_specs=[
        pl.BlockSpec((1, 3, 160, 128), lambda b: (b, 0, 0, 0)),
        pl.BlockSpec((1, 2, 160, 128), lambda b: (b, 0, 0, 0)),
        pl.BlockSpec((2, 160, 128), lambda b: (0, 0, 0)),
    ],
    out_specs=[
        pl.BlockSpec((1, 2, 160, 128), lambda b: (b, 0, 0, 0)),
        pl.BlockSpec((1, 160, 128), lambda b: (b, 0, 0)),
        pl.BlockSpec((1, 160, 128), lambda b: (b, 0, 0)),
    ],
    out_shape=[
        jax.ShapeDtypeStruct((_B, 2, 160, 128), jnp.float32),
        jax.ShapeDtypeStruct((_B, 160, 128), jnp.float32),
        jax.ShapeDtypeStruct((_B, 160, 128), jnp.float32),
    ],
)


@functools.partial(
    pl.kernel,
    out_type=jax.ShapeDtypeStruct((32, 3 * _KPAD), jnp.float32),
    mesh=plsc.VectorSubcoreMesh(
        core_axis_name="c", subcore_axis_name="s", num_cores=2, num_subcores=16
    ),
    compiler_params=pltpu.CompilerParams(needs_layout_passes=False),
    scratch_types=[
        pltpu.VMEM((_P,), jnp.float32),
        pltpu.VMEM((_P,), jnp.float32),
        pltpu.VMEM((_P,), jnp.float32),
        pltpu.VMEM((_P,), jnp.float32),
        pltpu.VMEM((_P,), jnp.float32),
        pltpu.VMEM((_P,), jnp.float32),
        pltpu.VMEM((_KPAD,), jnp.float32),
        pltpu.VMEM((_KPAD,), jnp.int32),
        pltpu.VMEM((_KPAD,), jnp.float32),
        pltpu.VMEM((_KPAD,), jnp.float32),
        pltpu.VMEM((_KPAD,), jnp.float32),
        pltpu.VMEM((_KPAD,), jnp.float32),
        pltpu.VMEM((_KPAD,), jnp.float32),
        pltpu.VMEM((3 * _KPAD,), jnp.float32),
    ],
)
def _select_nms(scores_hbm, starts_hbm, ends_hbm, out_hbm,
                sc_v, st_v, en_v, cv, cs, ce,
                tval, tpos, tst, ten, areas, supp, keep, ob):
    w = lax.axis_index("s") * 2 + lax.axis_index("c")
    b = w // 2
    pltpu.sync_copy(scores_hbm.at[w], sc_v)
    pltpu.sync_copy(starts_hbm.at[b], st_v)
    pltpu.sync_copy(ends_hbm.at[b], en_v)

    iota16 = lax.iota(jnp.int32, 16)
    lane0 = iota16 == 0
    zf = jnp.zeros((16,), jnp.float32)
    negf = jnp.full((16,), -1.0, jnp.float32)
    zi = jnp.zeros((16,), jnp.int32)

    def comp_body(i, cnt):
        base = i * 16
        v = sc_v[pl.ds(base, 16)]
        msk = v > _THR
        cum = plsc.cumsum(msk.astype(jnp.int32))
        pos = cum + (cnt - 1)
        plsc.store_scatter(cv, [pos], v, mask=msk)
        plsc.store_scatter(cs, [pos], st_v[pl.ds(base, 16)], mask=msk)
        plsc.store_scatter(ce, [pos], en_v[pl.ds(base, 16)], mask=msk)
        return cnt + cum[15]

    m_count = lax.fori_loop(0, _NCH, comp_body, jnp.int32(0))

    cv[pl.ds(m_count, 16)] = negf
    cs[pl.ds(m_count, 16)] = zf
    ce[pl.ds(m_count, 16)] = zf

    for j in range(_KPAD // 16):
        sl = pl.ds(j * 16, 16)
        tval[sl] = negf
        tpos[sl] = zi
        tst[sl] = zf
        ten[sl] = zf
        supp[sl] = zf
        keep[sl] = zf

    t_count = jnp.minimum(m_count, _K)
    nch = (m_count + 15) // 16
    big = jnp.int32(2 ** 30)

    def ext_body(k, _):
        def scan_body(j, carry):
            bv, bp = carry
            v = cv[pl.ds(j * 16, 16)]
            p = iota16 + j * 16
            better = v > bv
            return (jnp.where(better, v, bv), jnp.where(better, p, bp))

        bv, bp = lax.fori_loop(
            0, nch, scan_body,
            (jnp.full((16,), -2.0, jnp.float32), jnp.full((16,), big)),
        )
        m = jnp.max(bv)
        pos = jnp.min(jnp.where(bv == m, bp, big))
        ksplat = jnp.full((16,), k)
        plsc.store_scatter(tval, [ksplat], jnp.full((16,), m), mask=lane0)
        plsc.store_scatter(tpos, [ksplat], jnp.full((16,), pos), mask=lane0)
        plsc.store_scatter(cv, [jnp.full((16,), pos)], negf, mask=lane0)
        return 0

    lax.fori_loop(0, t_count, ext_body, 0)

    for j in range(_KPAD // 16):
        sl = pl.ds(j * 16, 16)
        idx = tpos[sl]
        tst[sl] = plsc.load_gather(cs, [idx])
        ten[sl] = plsc.load_gather(ce, [idx])
        areas[sl] = ten[sl] - tst[sl]

    def nms_body(i, _):
        isp = jnp.full((16,), i)
        sup_i = plsc.load_gather(supp, [isp])[0]
        val_i = plsc.load_gather(tval, [isp])[0]
        x_i = plsc.load_gather(tst, [isp])
        y_i = plsc.load_gather(ten, [isp])
        a_i = y_i - x_i
        is_keep = jnp.logical_and(sup_i == 0.0, val_i > _THR)
        kf = jnp.where(is_keep, 1.0, 0.0)
        plsc.store_scatter(keep, [isp], jnp.full((16,), kf), mask=lane0)
        for j in range(_KPAD // 16):
            sl = pl.ds(j * 16, 16)
            x = tst[sl]
            y = ten[sl]
            xx = jnp.maximum(x, x_i)
            yy = jnp.minimum(y, y_i)
            inter = jnp.maximum(yy - xx, 0.0)
            union = jnp.maximum(areas[sl] + a_i - inter, 1e-12)
            iou = inter / union
            gidx = iota16 + j * 16
            newly = jnp.logical_and(
                jnp.logical_and(iou > _OVR, gidx != i), is_keep
            )
            supp[sl] = jnp.where(newly, 1.0, supp[sl])
        return 0

    lax.fori_loop(0, _K, nms_body, 0)

    for j in range(_KPAD // 16):
        sl = pl.ds(j * 16, 16)
        kf = keep[sl] > 0.0
        ob[pl.ds(j * 16, 16)] = jnp.where(kf, tst[sl], 0.0)
        ob[pl.ds(_KPAD + j * 16, 16)] = jnp.where(kf, ten[sl], 0.0)
        ob[pl.ds(2 * _KPAD + j * 16, 16)] = jnp.where(kf, tval[sl], 0.0)
    pltpu.sync_copy(ob, out_hbm.at[w])


def kernel(localizations, classifications, localizations_default):
    pad = _P - _N
    cls_t = jnp.pad(jnp.transpose(classifications, (0, 2, 1)),
                    ((0, 0), (0, 0), (0, pad)))
    loc_t = jnp.pad(jnp.transpose(localizations, (0, 2, 1)),
                    ((0, 0), (0, 0), (0, pad)))
    dft_t = jnp.pad(localizations_default.T, ((0, 0), (0, pad)))

    scores, starts, ends = _dense(
        cls_t.reshape(_B, 3, 160, 128),
        loc_t.reshape(_B, 2, 160, 128),
        dft_t.reshape(2, 160, 128),
    )
    out = _select_nms(
        scores.reshape(2 * _B, _P),
        starts.reshape(_B, _P),
        ends.reshape(_B, _P),
    )
    out = out.reshape(32, 3, _KPAD)[:, :, :_K]
    return out.reshape(_B, 2, 3, _K).transpose(0, 1, 3, 2)


# speedup vs baseline: 445.5085x; 1.8381x over previous
import functools

import jax
import jax.numpy as jnp
from jax import lax
from jax.experimental import pallas as pl
from jax.experimental.pallas import tpu as pltpu, tpu_sc as plsc

_N = 20000
_P = 20480
_NCH = _P // 16
_B = 16
_K = 200
_KPAD = 208
_THR = 0.7
_OVR = 0.5
_NB = 32


def _dense_body(cls_ref, loc_ref, dft_ref, sc_ref, st_ref, en_ref):
    x0 = cls_ref[0, 0]
    x1 = cls_ref[0, 1]
    x2 = cls_ref[0, 2]
    m = jnp.maximum(x0, jnp.maximum(x1, x2))
    e0 = jnp.exp(x0 - m)
    e1 = jnp.exp(x1 - m)
    e2 = jnp.exp(x2 - m)
    s = e0 + e1 + e2
    sc_ref[0, 0] = e1 / s
    sc_ref[0, 1] = e2 / s
    l0 = loc_ref[0, 0]
    l1 = loc_ref[0, 1]
    d0 = dft_ref[0]
    d1 = dft_ref[1]
    centers = d0 + 0.1 * l0 * d1
    widths = d1 * jnp.exp(0.2 * l1)
    st = centers - widths / 2.0
    st_ref[0] = st
    en_ref[0] = st + widths


_dense = pl.pallas_call(
    _dense_body,
    grid=(_B,),
    in_specs=[
        pl.BlockSpec((1, 3, 160, 128), lambda b: (b, 0, 0, 0)),
        pl.BlockSpec((1, 2, 160, 128), lambda b: (b, 0, 0, 0)),
        pl.BlockSpec((2, 160, 128), lambda b: (0, 0, 0)),
    ],
    out_specs=[
        pl.BlockSpec((1, 2, 160, 128), lambda b: (b, 0, 0, 0)),
        pl.BlockSpec((1, 160, 128), lambda b: (b, 0, 0)),
        pl.BlockSpec((1, 160, 128), lambda b: (b, 0, 0)),
    ],
    out_shape=[
        jax.ShapeDtypeStruct((_B, 2, 160, 128), jnp.float32),
        jax.ShapeDtypeStruct((_B, 160, 128), jnp.float32),
        jax.ShapeDtypeStruct((_B, 160, 128), jnp.float32),
    ],
)


@functools.partial(
    pl.kernel,
    out_type=jax.ShapeDtypeStruct((32, 3 * _KPAD), jnp.float32),
    mesh=plsc.VectorSubcoreMesh(
        core_axis_name="c", subcore_axis_name="s", num_cores=2, num_subcores=16
    ),
    compiler_params=pltpu.CompilerParams(needs_layout_passes=False),
    scratch_types=[
        pltpu.VMEM((_P,), jnp.float32),
        pltpu.VMEM((_P,), jnp.float32),
        pltpu.VMEM((_P,), jnp.float32),
        pltpu.VMEM((_P,), jnp.float32),
        pltpu.VMEM((_P,), jnp.int32),
        pltpu.VMEM((_NB * 16,), jnp.int32),
        pltpu.VMEM((_KPAD,), jnp.float32),
        pltpu.VMEM((_KPAD,), jnp.int32),
        pltpu.VMEM((_KPAD,), jnp.float32),
        pltpu.VMEM((_KPAD,), jnp.float32),
        pltpu.VMEM((_KPAD,), jnp.float32),
        pltpu.VMEM((_KPAD,), jnp.float32),
        pltpu.VMEM((_KPAD,), jnp.float32),
        pltpu.VMEM((3 * _KPAD,), jnp.float32),
    ],
)
def _select_nms(scores_hbm, starts_hbm, ends_hbm, out_hbm,
                sc_v, st_v, en_v, cv, ci, hist,
                tval, tpos, tst, ten, areas, supp, keep, ob):
    w = lax.axis_index("s") * 2 + lax.axis_index("c")
    b = w // 2
    pltpu.sync_copy(scores_hbm.at[w], sc_v)
    pltpu.sync_copy(starts_hbm.at[b], st_v)
    pltpu.sync_copy(ends_hbm.at[b], en_v)

    iota16 = lax.iota(jnp.int32, 16)
    lane0 = iota16 == 0
    zf = jnp.zeros((16,), jnp.float32)
    negf = jnp.full((16,), -1.0, jnp.float32)
    zi = jnp.zeros((16,), jnp.int32)
    onei = jnp.full((16,), 1, jnp.int32)

    def comp_body(i, cnt):
        base = i * 16
        v = sc_v[pl.ds(base, 16)]
        msk = v > _THR
        cum = plsc.cumsum(msk.astype(jnp.int32))
        pos = cum + (cnt - 1)
        plsc.store_scatter(cv, [pos], v, mask=msk)
        plsc.store_scatter(ci, [pos], iota16 + base, mask=msk)
        return cnt + cum[15]

    m_count = lax.fori_loop(0, _NCH, comp_body, jnp.int32(0))

    cv[pl.ds(m_count, 16)] = negf
    ci[pl.ds(m_count, 16)] = zi

    for j in range(_NB):
        hist[pl.ds(j * 16, 16)] = zi
    nch = (m_count + 15) // 16
    binv = jnp.float32(_NB / 0.3)

    def hist_body(j, _):
        v = cv[pl.ds(j * 16, 16)]
        bk = jnp.clip((v - _THR) * binv, 0.0, _NB - 1.0).astype(jnp.int32)
        plsc.addupdate_scatter(hist, [iota16 * _NB + bk], onei, mask=v > _THR)
        return 0

    lax.fori_loop(0, nch, hist_body, 0)

    c0 = jnp.zeros((16,), jnp.int32)
    c1 = jnp.zeros((16,), jnp.int32)
    for s in range(16):
        c0 = c0 + hist[pl.ds(s * _NB, 16)]
        c1 = c1 + hist[pl.ds(s * _NB + 16, 16)]
    cum0 = plsc.cumsum(c0)
    cum1 = plsc.cumsum(c1)
    e0 = cum0 - c0
    e1 = cum1 - c1 + cum0[15]
    lim = m_count - _K
    negi = jnp.full((16,), -1, jnp.int32)
    k0 = jnp.max(jnp.where(e0 <= lim, iota16, negi))
    k1 = jnp.max(jnp.where(e1 <= lim, iota16 + 16, negi))
    selb = jnp.maximum(k0, k1)
    thr2 = jnp.where(
        selb >= 0,
        _THR + (selb.astype(jnp.float32) - 0.5) * jnp.float32(0.3 / _NB),
        0.0,
    )

    def rc_body(j, cnt):
        v = cv[pl.ds(j * 16, 16)]
        ii = ci[pl.ds(j * 16, 16)]
        msk = v > thr2
        cum = plsc.cumsum(msk.astype(jnp.int32))
        pos = cum + (cnt - 1)
        plsc.store_scatter(cv, [pos], v, mask=msk)
        plsc.store_scatter(ci, [pos], ii, mask=msk)
        return cnt + cum[15]

    m2 = lax.fori_loop(0, nch, rc_body, jnp.int32(0))
    cv[pl.ds(m2, 16)] = negf
    ci[pl.ds(m2, 16)] = zi

    for j in range(_KPAD // 16):
        sl = pl.ds(j * 16, 16)
        tval[sl] = negf
        tpos[sl] = zi
        keep[sl] = zf

    t_count = jnp.minimum(m_count, _K)
    nch2 = (m2 + 15) // 16
    big = jnp.int32(2 ** 30)

    def ext_body(k, _):
        def scan_body(j, carry):
            bv, bp = carry
            v = cv[pl.ds(j * 16, 16)]
            p = iota16 + j * 16
            better = v > bv
            return (jnp.where(better, v, bv), jnp.where(better, p, bp))

        bv, bp = lax.fori_loop(
            0, nch2, scan_body,
            (jnp.full((16,), -2.0, jnp.float32), jnp.full((16,), big)),
        )
        m = jnp.max(bv)
        pos = jnp.min(jnp.where(bv == m, bp, big))
        ksplat = jnp.full((16,), k)
        plsc.store_scatter(tval, [ksplat], jnp.full((16,), m), mask=lane0)
        plsc.store_scatter(tpos, [ksplat], jnp.full((16,), pos), mask=lane0)
        plsc.store_scatter(cv, [jnp.full((16,), pos)], negf, mask=lane0)
        return 0

    lax.fori_loop(0, t_count, ext_body, 0)

    for j in range(_KPAD // 16):
        sl = pl.ds(j * 16, 16)
        aidx = plsc.load_gather(ci, [tpos[sl]])
        x = plsc.load_gather(st_v, [aidx])
        y = plsc.load_gather(en_v, [aidx])
        tst[sl] = x
        ten[sl] = y
        areas[sl] = y - x
        supp[sl] = jnp.where(tval[sl] > _THR, 0.0, 1.0)

    def nms_body(i, _):
        isp = jnp.full((16,), i)
        sup_i = plsc.load_gather(supp, [isp])[0]

        @pl.when(sup_i == 0.0)
        def _():
            x_i = plsc.load_gather(tst, [isp])
            y_i = plsc.load_gather(ten, [isp])
            a_i = y_i - x_i
            plsc.store_scatter(keep, [isp], jnp.full((16,), 1.0), mask=lane0)
            for j in range(_KPAD // 16):
                sl = pl.ds(j * 16, 16)
                x = tst[sl]
                y = ten[sl]
                xx = jnp.maximum(x, x_i)
                yy = jnp.minimum(y, y_i)
                inter = jnp.maximum(yy - xx, 0.0)
                union = jnp.maximum(areas[sl] + a_i - inter, 1e-12)
                iou = inter / union
                gidx = iota16 + j * 16
                newly = jnp.logical_and(iou > _OVR, gidx != i)
                supp[sl] = jnp.where(newly, 1.0, supp[sl])

        return 0

    lax.fori_loop(0, _K, nms_body, 0)

    for j in range(_KPAD // 16):
        sl = pl.ds(j * 16, 16)
        kf = keep[sl] > 0.0
        ob[pl.ds(j * 16, 16)] = jnp.where(kf, tst[sl], 0.0)
        ob[pl.ds(_KPAD + j * 16, 16)] = jnp.where(kf, ten[sl], 0.0)
        ob[pl.ds(2 * _KPAD + j * 16, 16)] = jnp.where(kf, tval[sl], 0.0)
    pltpu.sync_copy(ob, out_hbm.at[w])


def kernel(localizations, classifications, localizations_default):
    pad = _P - _N
    cls_t = jnp.pad(jnp.transpose(classifications, (0, 2, 1)),
                    ((0, 0), (0, 0), (0, pad)))
    loc_t = jnp.pad(jnp.transpose(localizations, (0, 2, 1)),
                    ((0, 0), (0, 0), (0, pad)))
    dft_t = jnp.pad(localizations_default.T, ((0, 0), (0, pad)))

    scores, starts, ends = _dense(
        cls_t.reshape(_B, 3, 160, 128),
        loc_t.reshape(_B, 2, 160, 128),
        dft_t.reshape(2, 160, 128),
    )
    out = _select_nms(
        scores.reshape(2 * _B, _P),
        starts.reshape(_B, _P),
        ends.reshape(_B, _P),
    )
    out = out.reshape(32, 3, _KPAD)[:, :, :_K]
    return out.reshape(_B, 2, 3, _K).transpose(0, 1, 3, 2)


# speedup vs baseline: 490.4686x; 1.1009x over previous
import functools

import jax
import jax.numpy as jnp
from jax import lax
from jax.experimental import pallas as pl
from jax.experimental.pallas import tpu as pltpu, tpu_sc as plsc

_N = 20000
_P = 20480
_NCH = _N // 16
_B = 16
_K = 200
_KPAD = 208
_THR = 0.7
_OVR = 0.5
_NB = 32


def _dense_body(cls_ref, loc_ref, dft_ref, sc_ref, st_ref, en_ref):
    x0 = cls_ref[0, 0:1, :]
    x1 = cls_ref[0, 1:2, :]
    x2 = cls_ref[0, 2:3, :]
    m = jnp.maximum(x0, jnp.maximum(x1, x2))
    e0 = jnp.exp(x0 - m)
    e1 = jnp.exp(x1 - m)
    e2 = jnp.exp(x2 - m)
    s = e0 + e1 + e2
    sc_ref[0, 0:1, :] = e1 / s
    sc_ref[0, 1:2, :] = e2 / s
    l0 = loc_ref[0, 0:1, :]
    l1 = loc_ref[0, 1:2, :]
    d0 = dft_ref[0:1, :]
    d1 = dft_ref[1:2, :]
    centers = d0 + 0.1 * l0 * d1
    widths = d1 * jnp.exp(0.2 * l1)
    st = centers - widths / 2.0
    st_ref[0] = st
    en_ref[0] = st + widths


_dense = pl.pallas_call(
    _dense_body,
    grid=(_B,),
    in_specs=[
        pl.BlockSpec((1, 3, _N), lambda b: (b, 0, 0)),
        pl.BlockSpec((1, 2, _N), lambda b: (b, 0, 0)),
        pl.BlockSpec((2, _N), lambda b: (0, 0)),
    ],
    out_specs=[
        pl.BlockSpec((1, 2, _N), lambda b: (b, 0, 0)),
        pl.BlockSpec((1, 1, _N), lambda b: (b, 0, 0)),
        pl.BlockSpec((1, 1, _N), lambda b: (b, 0, 0)),
    ],
    out_shape=[
        jax.ShapeDtypeStruct((_B, 2, _N), jnp.float32),
        jax.ShapeDtypeStruct((_B, 1, _N), jnp.float32),
        jax.ShapeDtypeStruct((_B, 1, _N), jnp.float32),
    ],
)


@functools.partial(
    pl.kernel,
    out_type=jax.ShapeDtypeStruct((32, 3 * _KPAD), jnp.float32),
    mesh=plsc.VectorSubcoreMesh(
        core_axis_name="c", subcore_axis_name="s", num_cores=2, num_subcores=16
    ),
    compiler_params=pltpu.CompilerParams(needs_layout_passes=False),
    scratch_types=[
        pltpu.VMEM((_N,), jnp.float32),
        pltpu.VMEM((_N,), jnp.float32),
        pltpu.VMEM((_N,), jnp.float32),
        pltpu.VMEM((_P,), jnp.float32),
        pltpu.VMEM((_P,), jnp.int32),
        pltpu.VMEM((_NB * 16,), jnp.int32),
        pltpu.VMEM((_KPAD,), jnp.float32),
        pltpu.VMEM((_KPAD,), jnp.int32),
        pltpu.VMEM((_KPAD,), jnp.float32),
        pltpu.VMEM((_KPAD,), jnp.float32),
        pltpu.VMEM((_KPAD,), jnp.float32),
        pltpu.VMEM((_KPAD,), jnp.float32),
        pltpu.VMEM((_KPAD,), jnp.float32),
        pltpu.VMEM((3 * _KPAD,), jnp.float32),
    ],
)
def _select_nms(scores_hbm, starts_hbm, ends_hbm, out_hbm,
                sc_v, st_v, en_v, cv, ci, hist,
                tval, tpos, tst, ten, areas, supp, keep, ob):
    w = lax.axis_index("s") * 2 + lax.axis_index("c")
    b = w // 2
    pltpu.sync_copy(scores_hbm.at[w], sc_v)
    pltpu.sync_copy(starts_hbm.at[b], st_v)
    pltpu.sync_copy(ends_hbm.at[b], en_v)

    iota16 = lax.iota(jnp.int32, 16)
    lane0 = iota16 == 0
    zf = jnp.zeros((16,), jnp.float32)
    negf = jnp.full((16,), -1.0, jnp.float32)
    zi = jnp.zeros((16,), jnp.int32)
    onei = jnp.full((16,), 1, jnp.int32)

    def comp_body(i, cnt):
        base = i * 16
        v = sc_v[pl.ds(base, 16)]
        msk = v > _THR
        plsc.store_compressed(cv.at[pl.ds(cnt, 16)], v, mask=msk)
        plsc.store_compressed(ci.at[pl.ds(cnt, 16)], iota16 + base, mask=msk)
        return cnt + plsc.all_reduce_population_count(msk)[0]

    m_count = lax.fori_loop(0, _NCH, comp_body, jnp.int32(0))

    cv[pl.ds(m_count, 16)] = negf
    ci[pl.ds(m_count, 16)] = zi

    for j in range(_NB):
        hist[pl.ds(j * 16, 16)] = zi
    nch = (m_count + 15) // 16
    binv = jnp.float32(_NB / 0.3)

    def hist_body(j, _):
        v = cv[pl.ds(j * 16, 16)]
        bk = jnp.clip((v - _THR) * binv, 0.0, _NB - 1.0).astype(jnp.int32)
        plsc.addupdate_scatter(hist, [iota16 * _NB + bk], onei, mask=v > _THR)
        return 0

    lax.fori_loop(0, nch, hist_body, 0)

    c0 = jnp.zeros((16,), jnp.int32)
    c1 = jnp.zeros((16,), jnp.int32)
    for s in range(16):
        c0 = c0 + hist[pl.ds(s * _NB, 16)]
        c1 = c1 + hist[pl.ds(s * _NB + 16, 16)]
    cum0 = plsc.cumsum(c0)
    cum1 = plsc.cumsum(c1)
    e0 = cum0 - c0
    e1 = cum1 - c1 + cum0[15]
    lim = m_count - _K
    negi = jnp.full((16,), -1, jnp.int32)
    k0 = jnp.max(jnp.where(e0 <= lim, iota16, negi))
    k1 = jnp.max(jnp.where(e1 <= lim, iota16 + 16, negi))
    selb = jnp.maximum(k0, k1)
    thr2 = jnp.where(
        selb >= 0,
        _THR + (selb.astype(jnp.float32) - 0.5) * jnp.float32(0.3 / _NB),
        0.0,
    )

    def rc_body(j, cnt):
        v = cv[pl.ds(j * 16, 16)]
        ii = ci[pl.ds(j * 16, 16)]
        msk = v > thr2
        plsc.store_compressed(cv.at[pl.ds(cnt, 16)], v, mask=msk)
        plsc.store_compressed(ci.at[pl.ds(cnt, 16)], ii, mask=msk)
        return cnt + plsc.all_reduce_population_count(msk)[0]

    m2 = lax.fori_loop(0, nch, rc_body, jnp.int32(0))
    cv[pl.ds(m2, 16)] = negf
    ci[pl.ds(m2, 16)] = zi

    for j in range(_KPAD // 16):
        sl = pl.ds(j * 16, 16)
        tval[sl] = negf
        tpos[sl] = zi
        keep[sl] = zf

    t_count = jnp.minimum(m_count, _K)
    nch2 = (m2 + 15) // 16
    big = jnp.int32(2 ** 30)

    def ext_body(k, _):
        def scan_body(j, carry):
            bv, bp = carry
            v = cv[pl.ds(j * 16, 16)]
            p = iota16 + j * 16
            better = v > bv
            return (jnp.where(better, v, bv), jnp.where(better, p, bp))

        bv, bp = lax.fori_loop(
            0, nch2, scan_body,
            (jnp.full((16,), -2.0, jnp.float32), jnp.full((16,), big)),
        )
        m = jnp.max(bv)
        pos = jnp.min(jnp.where(bv == m, bp, big))
        ksplat = jnp.full((16,), k)
        plsc.store_scatter(tval, [ksplat], jnp.full((16,), m), mask=lane0)
        plsc.store_scatter(tpos, [ksplat], jnp.full((16,), pos), mask=lane0)
        plsc.store_scatter(cv, [jnp.full((16,), pos)], negf, mask=lane0)
        return 0

    lax.fori_loop(0, t_count, ext_body, 0)

    for j in range(_KPAD // 16):
        sl = pl.ds(j * 16, 16)
        aidx = plsc.load_gather(ci, [tpos[sl]])
        x = plsc.load_gather(st_v, [aidx])
        y = plsc.load_gather(en_v, [aidx])
        tst[sl] = x
        ten[sl] = y
        areas[sl] = y - x
        supp[sl] = jnp.where(tval[sl] > _THR, 0.0, 1.0)

    def nms_body(i, _):
        isp = jnp.full((16,), i)
        sup_i = plsc.load_gather(supp, [isp])[0]

        @pl.when(sup_i == 0.0)
        def _():
            x_i = plsc.load_gather(tst, [isp])
            y_i = plsc.load_gather(ten, [isp])
            a_i = y_i - x_i
            plsc.store_scatter(keep, [isp], jnp.full((16,), 1.0), mask=lane0)
            for j in range(_KPAD // 16):
                sl = pl.ds(j * 16, 16)
                x = tst[sl]
                y = ten[sl]
                xx = jnp.maximum(x, x_i)
                yy = jnp.minimum(y, y_i)
                inter = jnp.maximum(yy - xx, 0.0)
                union = jnp.maximum(areas[sl] + a_i - inter, 1e-12)
                iou = inter / union
                gidx = iota16 + j * 16
                newly = jnp.logical_and(iou > _OVR, gidx != i)
                supp[sl] = jnp.where(newly, 1.0, supp[sl])

        return 0

    lax.fori_loop(0, _K, nms_body, 0)

    for j in range(_KPAD // 16):
        sl = pl.ds(j * 16, 16)
        kf = keep[sl] > 0.0
        ob[pl.ds(j * 16, 16)] = jnp.where(kf, tst[sl], 0.0)
        ob[pl.ds(_KPAD + j * 16, 16)] = jnp.where(kf, ten[sl], 0.0)
        ob[pl.ds(2 * _KPAD + j * 16, 16)] = jnp.where(kf, tval[sl], 0.0)
    pltpu.sync_copy(ob, out_hbm.at[w])


def kernel(localizations, classifications, localizations_default):
    cls_t = jnp.transpose(classifications, (0, 2, 1))
    loc_t = jnp.transpose(localizations, (0, 2, 1))
    dft_t = localizations_default.T

    scores, starts, ends = _dense(cls_t, loc_t, dft_t)
    out = _select_nms(
        scores.reshape(2 * _B, _N),
        starts.reshape(_B, _N),
        ends.reshape(_B, _N),
    )
    out = out.reshape(32, 3, _KPAD)[:, :, :_K]
    return out.reshape(_B, 2, 3, _K).transpose(0, 1, 3, 2)
